# baseline (speedup 1.0000x reference)
import math

import jax
import jax.numpy as jnp
from jax import lax
from jax.experimental import pallas as pl
from jax.experimental.pallas import tpu as pltpu
import jax.experimental.pallas.tpu_sc as plsc

H = 128
HC = 64
B = 256
BD = 272
NC = 2
NS = 16
L = 16
RB = 96
NG = HC // L
NEG = -3.0e38


def _make_sc_body(n):
  nfull = n // RB
  tail = n - nfull * RB
  nblk = nfull + (1 if tail else 0)
  kmax = (nblk + NS - 1) // NS
  assert tail % L == 0

  def _sc_body(h2_hbm, batch_hbm, stats_out,
               rowbuf0, rowbuf1, idxg0, idxg1, sumacc, sqacc, maxacc,
               prevbuf, hsem0, hsem1, isem0, isem1):
    cid = lax.axis_index("c")
    sid = lax.axis_index("s")
    rowbufs = (rowbuf0, rowbuf1)
    idxgs = (idxg0, idxg1)
    hsems = (hsem0, hsem1)
    isems = (isem0, isem1)

    zv = jnp.zeros((L,), jnp.float32)
    nv = jnp.full((L,), NEG, jnp.float32)

    def fillrow(i, _):
      for g in range(8):
        sl = pl.ds(g * L, L)
        sumacc[i, sl] = zv
        sqacc[i, sl] = zv
        maxacc[i, sl] = nv
      return 0
    lax.fori_loop(0, BD, fillrow, 0)

    RR = B + 2

    def run(i):
      return maxacc[RR + i, pl.ds(0, L)]

    def setrun(i, val):
      maxacc[RR + i, pl.ds(0, L)] = val

    def flush(pv):
      cs = pl.ds(HC, L)
      sumacc[pv, cs] = sumacc[pv, cs] + run(0)
      for g in range(NG):
        sl = pl.ds(g * L, L)
        sumacc[pv, sl] = sumacc[pv, sl] + run(1 + g)
        sqacc[pv, sl] = sqacc[pv, sl] + run(5 + g)
        maxacc[pv, sl] = jnp.maximum(maxacc[pv, sl], run(9 + g))

    def reset_run():
      for i in range(9):
        setrun(i, zv)
      for i in range(9, 13):
        setrun(i, nv)

    reset_run()
    prevbuf[0] = jnp.int32(-1)

    def block_flags(knum):
      blk = sid + NS * knum
      in_r = blk < nblk
      is_l = (blk == nblk - 1) if tail else jnp.bool_(False)
      return blk, in_r, is_l

    def copies(blk, b, full):
      rows = RB if full else tail
      start = blk * RB if full else nfull * RB
      return (
          pltpu.make_async_copy(h2_hbm.at[pl.ds(start, rows)],
                                rowbufs[b].at[pl.ds(0, rows)], hsems[b]),
          pltpu.make_async_copy(batch_hbm.at[pl.ds(start, rows)],
                                idxgs[b].at[pl.ds(0, rows)], isems[b]),
      )

    def issue(knum, b):
      blk, in_r, is_l = block_flags(knum)

      @pl.when(in_r & jnp.logical_not(is_l))
      def _():
        for c in copies(blk, b, True):
          c.start()

      if tail:
        @pl.when(is_l)
        def _():
          for c in copies(blk, b, False):
            c.start()

    def wait_for(knum, b):
      blk, in_r, is_l = block_flags(knum)

      @pl.when(in_r & jnp.logical_not(is_l))
      def _():
        for c in copies(blk, b, True):
          c.wait()

      if tail:
        @pl.when(is_l)
        def _():
          for c in copies(blk, b, False):
            c.wait()
          for off in range(tail, RB, L):
            idxgs[b][pl.ds(off, L)] = jnp.full((L,), B, jnp.int32)

      @pl.when(jnp.logical_not(in_r))
      def _():
        for c in range(RB // L):
          idxgs[b][pl.ds(c * L, L)] = jnp.full((L,), B, jnp.int32)

    def compute(b):
      rowbuf = rowbufs[b]
      idxg = idxgs[b]

      def chunk_body(c, _):
        idvec = idxg[pl.ds(c * L, L)]
        prev = prevbuf[0]
        fast = (idvec[0] == prev) & (idvec[L - 1] == prev)

        @pl.when(fast)
        def _():
          sums = [run(1 + g) for g in range(NG)]
          sqs = [run(5 + g) for g in range(NG)]
          mxs = [run(9 + g) for g in range(NG)]
          for j in range(L):
            for g in range(NG):
              v = rowbuf[c * L + j, pl.ds(cid * HC + g * L, L)]
              sums[g] = sums[g] + v
              sqs[g] = sqs[g] + v * v
              mxs[g] = jnp.maximum(mxs[g], v)
          setrun(0, run(0) + 16.0)
          for g in range(NG):
            setrun(1 + g, sums[g])
            setrun(5 + g, sqs[g])
            setrun(9 + g, mxs[g])

        @pl.when(jnp.logical_not(fast))
        def _():
          for j in range(L):
            sj = idvec[j]
            pv = prevbuf[0]
            changed = sj != pv

            @pl.when((pv >= 0) & changed)
            def _():
              flush(pv)

            @pl.when(changed)
            def _():
              reset_run()
              prevbuf[0] = sj

            setrun(0, run(0) + 1.0)
            for g in range(NG):
              v = rowbuf[c * L + j, pl.ds(cid * HC + g * L, L)]
              setrun(1 + g, run(1 + g) + v)
              setrun(5 + g, run(5 + g) + v * v)
              setrun(9 + g, jnp.maximum(run(9 + g), v))
        return 0

      lax.fori_loop(0, RB // L, chunk_body, 0)

    issue(0, 0)

    def pair_body(i, _):
      k0 = 2 * i
      wait_for(k0, 0)
      issue(k0 + 1, 1)
      compute(0)
      wait_for(k0 + 1, 1)
      issue(k0 + 2, 0)
      compute(1)
      return 0

    lax.fori_loop(0, (kmax + 1) // 2, pair_body, 0)

    pvf = prevbuf[0]

    @pl.when(pvf >= 0)
    def _():
      flush(pvf)

    pltpu.sync_copy(sumacc.at[pl.ds(0, B)], stats_out.at[cid, sid, 0])
    pltpu.sync_copy(sqacc.at[pl.ds(0, B)], stats_out.at[cid, sid, 1])
    pltpu.sync_copy(maxacc.at[pl.ds(0, B)], stats_out.at[cid, sid, 2])

  return _sc_body


def _pool_sc(h, batch, interpret=False):
  n = h.shape[0]
  mesh = plsc.VectorSubcoreMesh(core_axis_name="c", subcore_axis_name="s",
                                num_cores=NC, num_subcores=NS)
  f = pl.kernel(
      _make_sc_body(n),
      out_type=[
          jax.ShapeDtypeStruct((NC, NS, 3, B, H), jnp.float32),
      ],
      mesh=mesh,
      interpret=interpret,
      scratch_types=[
          pltpu.VMEM((RB, H), jnp.float32),
          pltpu.VMEM((RB, H), jnp.float32),
          pltpu.VMEM((RB,), jnp.int32),
          pltpu.VMEM((RB,), jnp.int32),
          pltpu.VMEM((BD, H), jnp.float32),
          pltpu.VMEM((BD, H), jnp.float32),
          pltpu.VMEM((BD, H), jnp.float32),
          pltpu.SMEM((1,), jnp.int32),
          pltpu.SemaphoreType.DMA,
          pltpu.SemaphoreType.DMA,
          pltpu.SemaphoreType.DMA,
          pltpu.SemaphoreType.DMA,
      ],
  )
  return f(h, batch)[0]


def _tc_body(stats_ref, w1_ref, b1_ref, w2_ref, b2_ref, out_ref):
  st = stats_ref[...]
  s0 = jnp.sum(st[0, :, 0], axis=0)
  s1 = jnp.sum(st[1, :, 0], axis=0)
  q0 = jnp.sum(st[0, :, 1], axis=0)
  q1 = jnp.sum(st[1, :, 1], axis=0)
  m0 = jnp.max(st[0, :, 2], axis=0)
  m1 = jnp.max(st[1, :, 2], axis=0)
  ssum = jnp.concatenate([s0[:, :HC], s1[:, :HC]], axis=1)
  ssq = jnp.concatenate([q0[:, :HC], q1[:, :HC]], axis=1)
  smax = jnp.concatenate([m0[:, :HC], m1[:, :HC]], axis=1)
  count = s0[:, HC]
  safe = jnp.maximum(count, 1.0)[:, None]
  mean = ssum / safe
  var = jnp.maximum(ssq / safe - mean * mean, 0.0)
  std = jnp.sqrt(var + 1e-8)
  smax = jnp.where(count[:, None] > 0.0, smax, 0.0)
  g = jnp.concatenate([mean, smax, std], axis=1)
  hid = jax.nn.relu(
      jnp.dot(g, w1_ref[...], preferred_element_type=jnp.float32)
      + b1_ref[...])
  z = jnp.tanh(
      jnp.dot(hid, w2_ref[...], preferred_element_type=jnp.float32)
      + b2_ref[...]) * math.pi
  out_ref[...] = z


def _head_tc(stats, W1, b1, W2, b2, interpret=False):
  w2p = jnp.zeros((32, 128), jnp.float32).at[:, :8].set(W2)
  b2p = jnp.zeros((1, 128), jnp.float32).at[:, :8].set(b2)
  out = pl.pallas_call(
      _tc_body,
      out_shape=jax.ShapeDtypeStruct((B, 128), jnp.float32),
      interpret=interpret,
  )(stats, W1, b1.reshape(1, 32), w2p, b2p)
  return out[:, :8]


def kernel(h, batch, W1, b1, W2, b2):
  stats = _pool_sc(h, batch)
  return _head_tc(stats, W1, b1, W2, b2)


# speedup vs baseline: 1.6069x; 1.6069x over previous
import math

import jax
import jax.numpy as jnp
from jax import lax
from jax.experimental import pallas as pl
from jax.experimental.pallas import tpu as pltpu
import jax.experimental.pallas.tpu_sc as plsc

H = 128
HC = 64
B = 256
BD = 272
NC = 2
NS = 16
L = 16
RB = 128
NG = HC // L
NEG = -3.0e38


def _make_sc_body(n):
  nfull = n // RB
  tail = n - nfull * RB
  nblk = nfull + (1 if tail else 0)
  kmax = (nblk + NS - 1) // NS
  assert tail % L == 0

  def _sc_body(h2_hbm, batch_hbm, stats_out,
               rowbuf, idxg, sumacc, sqacc, maxacc):
    cid = lax.axis_index("c")
    sid = lax.axis_index("s")

    zv = jnp.zeros((L,), jnp.float32)
    nv = jnp.full((L,), NEG, jnp.float32)

    def fillrow(i, _):
      for g in range(8):
        sl = pl.ds(g * L, L)
        sumacc[i, sl] = zv
        sqacc[i, sl] = zv
        maxacc[i, sl] = nv
      return 0
    lax.fori_loop(0, BD, fillrow, 0)

    def flush(prev, rcnt, rsum, rsq, rmax):
      cs = pl.ds(HC, L)
      sumacc[prev, cs] = sumacc[prev, cs] + rcnt
      for g in range(NG):
        sl = pl.ds(g * L, L)
        sumacc[prev, sl] = sumacc[prev, sl] + rsum[g]
        sqacc[prev, sl] = sqacc[prev, sl] + rsq[g]
        maxacc[prev, sl] = jnp.maximum(maxacc[prev, sl], rmax[g])

    zeroN = tuple(jnp.zeros((L,), jnp.float32) for _ in range(NG))
    negN = tuple(jnp.full((L,), NEG, jnp.float32) for _ in range(NG))

    def blk_body(k, carry):
      blk = sid + NS * k
      in_range = blk < nblk
      is_last = blk == (nblk - 1) if tail else jnp.bool_(False)

      @pl.when(in_range & jnp.logical_not(is_last))
      def _():
        pltpu.sync_copy(h2_hbm.at[pl.ds(blk * RB, RB)], rowbuf)
        pltpu.sync_copy(batch_hbm.at[pl.ds(blk * RB, RB)],
                        idxg.at[pl.ds(0, RB)])

      if tail:
        @pl.when(is_last)
        def _():
          pltpu.sync_copy(h2_hbm.at[pl.ds(nfull * RB, tail)],
                          rowbuf.at[pl.ds(0, tail)])
          pltpu.sync_copy(batch_hbm.at[pl.ds(nfull * RB, tail)],
                          idxg.at[pl.ds(0, tail)])
          for off in range(tail, RB, L):
            idxg[pl.ds(off, L)] = jnp.full((L,), B, jnp.int32)

      @pl.when(jnp.logical_not(in_range))
      def _():
        for c in range(RB // L):
          idxg[pl.ds(c * L, L)] = jnp.full((L,), B, jnp.int32)

      def chunk_body(c, rc):
        idvec = idxg[pl.ds(c * L, L)]
        cc = rc
        for j in range(L):
          prevj = cc[0]
          rcnt = cc[1]
          rsum = cc[2:2 + NG]
          rsq = cc[2 + NG:2 + 2 * NG]
          rmax = cc[2 + 2 * NG:2 + 3 * NG]
          s = idvec[j]
          changed = s != prevj

          @pl.when((prevj >= 0) & changed)
          def _():
            flush(prevj, rcnt, rsum, rsq, rmax)

          st = lax.cond(
              changed,
              lambda: (jnp.zeros((L,), jnp.float32),) + zeroN + zeroN + negN,
              lambda rcnt=rcnt, rsum=rsum, rsq=rsq, rmax=rmax:
                  (rcnt,) + tuple(rsum) + tuple(rsq) + tuple(rmax))
          ncnt = st[0] + 1.0
          nsum, nsq, nmax = [], [], []
          for g in range(NG):
            v = rowbuf[c * L + j, pl.ds(cid * HC + g * L, L)]
            nsum.append(st[1 + g] + v)
            nsq.append(st[1 + NG + g] + v * v)
            nmax.append(jnp.maximum(st[1 + 2 * NG + g], v))
          cc = (s, ncnt) + tuple(nsum) + tuple(nsq) + tuple(nmax)
        return cc

      return lax.fori_loop(0, RB // L, chunk_body, carry)

    init = (jnp.int32(-1), jnp.zeros((L,), jnp.float32)) + zeroN + zeroN + negN
    carry = lax.fori_loop(0, kmax, blk_body, init)

    @pl.when(carry[0] >= 0)
    def _():
      flush(carry[0], carry[1], carry[2:2 + NG], carry[2 + NG:2 + 2 * NG],
            carry[2 + 2 * NG:2 + 3 * NG])

    pltpu.sync_copy(sumacc.at[pl.ds(0, B)], stats_out.at[cid, sid, 0])
    pltpu.sync_copy(sqacc.at[pl.ds(0, B)], stats_out.at[cid, sid, 1])
    pltpu.sync_copy(maxacc.at[pl.ds(0, B)], stats_out.at[cid, sid, 2])

  return _sc_body


def _pool_sc(h, batch, interpret=False):
  n = h.shape[0]
  mesh = plsc.VectorSubcoreMesh(core_axis_name="c", subcore_axis_name="s",
                                num_cores=NC, num_subcores=NS)
  f = pl.kernel(
      _make_sc_body(n),
      out_type=[
          jax.ShapeDtypeStruct((NC, NS, 3, B, H), jnp.float32),
      ],
      mesh=mesh,
      interpret=interpret,
      scratch_types=[
          pltpu.VMEM((RB, H), jnp.float32),
          pltpu.VMEM((RB,), jnp.int32),
          pltpu.VMEM((BD, H), jnp.float32),
          pltpu.VMEM((BD, H), jnp.float32),
          pltpu.VMEM((BD, H), jnp.float32),
      ],
  )
  return f(h, batch)[0]


def _tc_body(stats_ref, w1_ref, b1_ref, w2_ref, b2_ref, out_ref):
  st = stats_ref[...]
  s0 = jnp.sum(st[0, :, 0], axis=0)
  s1 = jnp.sum(st[1, :, 0], axis=0)
  q0 = jnp.sum(st[0, :, 1], axis=0)
  q1 = jnp.sum(st[1, :, 1], axis=0)
  m0 = jnp.max(st[0, :, 2], axis=0)
  m1 = jnp.max(st[1, :, 2], axis=0)
  ssum = jnp.concatenate([s0[:, :HC], s1[:, :HC]], axis=1)
  ssq = jnp.concatenate([q0[:, :HC], q1[:, :HC]], axis=1)
  smax = jnp.concatenate([m0[:, :HC], m1[:, :HC]], axis=1)
  count = s0[:, HC]
  safe = jnp.maximum(count, 1.0)[:, None]
  mean = ssum / safe
  var = jnp.maximum(ssq / safe - mean * mean, 0.0)
  std = jnp.sqrt(var + 1e-8)
  smax = jnp.where(count[:, None] > 0.0, smax, 0.0)
  g = jnp.concatenate([mean, smax, std], axis=1)
  hid = jax.nn.relu(
      jnp.dot(g, w1_ref[...], preferred_element_type=jnp.float32)
      + b1_ref[...])
  z = jnp.tanh(
      jnp.dot(hid, w2_ref[...], preferred_element_type=jnp.float32)
      + b2_ref[...]) * math.pi
  out_ref[...] = z


def _head_tc(stats, W1, b1, W2, b2, interpret=False):
  w2p = jnp.zeros((32, 128), jnp.float32).at[:, :8].set(W2)
  b2p = jnp.zeros((1, 128), jnp.float32).at[:, :8].set(b2)
  out = pl.pallas_call(
      _tc_body,
      out_shape=jax.ShapeDtypeStruct((B, 128), jnp.float32),
      interpret=interpret,
  )(stats, W1, b1.reshape(1, 32), w2p, b2p)
  return out[:, :8]


def kernel(h, batch, W1, b1, W2, b2):
  stats = _pool_sc(h, batch)
  return _head_tc(stats, W1, b1, W2, b2)


# speedup vs baseline: 2.5898x; 1.6117x over previous
import math

import jax
import jax.numpy as jnp
from jax import lax
from jax.experimental import pallas as pl
from jax.experimental.pallas import tpu as pltpu
import jax.experimental.pallas.tpu_sc as plsc

H = 128
HC = 64
B = 256
BD = 272
NC = 2
NS = 16
L = 16
RB = 96
NG = HC // L
NEG = -3.0e38


def _make_sc_body(n):
  nfull = n // RB
  tail = n - nfull * RB
  nblk = nfull + (1 if tail else 0)
  kmax = (nblk + NS - 1) // NS
  assert tail % L == 0

  def _sc_body(h2_hbm, batch_hbm, stats_out,
               rowbuf0, rowbuf1, idxg0, idxg1, sumacc, sqacc, maxacc,
               hsem0, hsem1, isem0, isem1):
    cid = lax.axis_index("c")
    sid = lax.axis_index("s")
    rowbufs = (rowbuf0, rowbuf1)
    idxgs = (idxg0, idxg1)
    hsems = (hsem0, hsem1)
    isems = (isem0, isem1)

    zv = jnp.zeros((L,), jnp.float32)
    nv = jnp.full((L,), NEG, jnp.float32)

    def fillrow(i, _):
      for g in range(8):
        sl = pl.ds(g * L, L)
        sumacc[i, sl] = zv
        sqacc[i, sl] = zv
        maxacc[i, sl] = nv
      return 0
    lax.fori_loop(0, BD, fillrow, 0)

    def flush(prev, rcnt, rsum, rsq, rmax):
      cs = pl.ds(HC, L)
      sumacc[prev, cs] = sumacc[prev, cs] + rcnt
      for g in range(NG):
        sl = pl.ds(g * L, L)
        sumacc[prev, sl] = sumacc[prev, sl] + rsum[g]
        sqacc[prev, sl] = sqacc[prev, sl] + rsq[g]
        maxacc[prev, sl] = jnp.maximum(maxacc[prev, sl], rmax[g])

    zeroN = tuple(jnp.zeros((L,), jnp.float32) for _ in range(NG))
    negN = tuple(jnp.full((L,), NEG, jnp.float32) for _ in range(NG))

    def block_flags(knum):
      blk = sid + NS * knum
      in_r = blk < nblk
      is_l = (blk == nblk - 1) if tail else jnp.bool_(False)
      return blk, in_r, is_l

    def copies(blk, b, full):
      rows = RB if full else tail
      start = blk * RB if full else nfull * RB
      return (
          pltpu.make_async_copy(h2_hbm.at[pl.ds(start, rows)],
                                rowbufs[b].at[pl.ds(0, rows)], hsems[b]),
          pltpu.make_async_copy(batch_hbm.at[pl.ds(start, rows)],
                                idxgs[b].at[pl.ds(0, rows)], isems[b]),
      )

    def issue(knum, b):
      blk, in_r, is_l = block_flags(knum)

      @pl.when(in_r & jnp.logical_not(is_l))
      def _():
        for c in copies(blk, b, True):
          c.start()

      if tail:
        @pl.when(is_l)
        def _():
          for c in copies(blk, b, False):
            c.start()

    def wait_for(knum, b):
      blk, in_r, is_l = block_flags(knum)

      @pl.when(in_r & jnp.logical_not(is_l))
      def _():
        for c in copies(blk, b, True):
          c.wait()

      if tail:
        @pl.when(is_l)
        def _():
          for c in copies(blk, b, False):
            c.wait()
          for off in range(tail, RB, L):
            idxgs[b][pl.ds(off, L)] = jnp.full((L,), B, jnp.int32)

      @pl.when(jnp.logical_not(in_r))
      def _():
        for c in range(RB // L):
          idxgs[b][pl.ds(c * L, L)] = jnp.full((L,), B, jnp.int32)

    def compute(b, carry):
      rowbuf = rowbufs[b]
      idxg = idxgs[b]

      def chunk_body(c, rc):
        idvec = idxg[pl.ds(c * L, L)]
        cc = rc
        for j in range(L):
          prevj = cc[0]
          rcnt = cc[1]
          rsum = cc[2:2 + NG]
          rsq = cc[2 + NG:2 + 2 * NG]
          rmax = cc[2 + 2 * NG:2 + 3 * NG]
          s = idvec[j]
          changed = s != prevj

          @pl.when((prevj >= 0) & changed)
          def _():
            flush(prevj, rcnt, rsum, rsq, rmax)

          st = lax.cond(
              changed,
              lambda: (jnp.zeros((L,), jnp.float32),) + zeroN + zeroN + negN,
              lambda rcnt=rcnt, rsum=rsum, rsq=rsq, rmax=rmax:
                  (rcnt,) + tuple(rsum) + tuple(rsq) + tuple(rmax))
          ncnt = st[0] + 1.0
          nsum, nsq, nmax = [], [], []
          for g in range(NG):
            v = rowbuf[c * L + j, pl.ds(cid * HC + g * L, L)]
            nsum.append(st[1 + g] + v)
            nsq.append(st[1 + NG + g] + v * v)
            nmax.append(jnp.maximum(st[1 + 2 * NG + g], v))
          cc = (s, ncnt) + tuple(nsum) + tuple(nsq) + tuple(nmax)
        return cc

      return lax.fori_loop(0, RB // L, chunk_body, carry)

    issue(0, 0)

    def pair_body(i, carry):
      k0 = 2 * i
      wait_for(k0, 0)
      issue(k0 + 1, 1)
      carry = compute(0, carry)
      wait_for(k0 + 1, 1)
      issue(k0 + 2, 0)
      carry = compute(1, carry)
      return carry

    init = (jnp.int32(-1), jnp.zeros((L,), jnp.float32)) + zeroN + zeroN + negN
    carry = lax.fori_loop(0, (kmax + 1) // 2, pair_body, init)

    @pl.when(carry[0] >= 0)
    def _():
      flush(carry[0], carry[1], carry[2:2 + NG], carry[2 + NG:2 + 2 * NG],
            carry[2 + 2 * NG:2 + 3 * NG])

    pltpu.sync_copy(sumacc.at[pl.ds(0, B)], stats_out.at[cid, sid, 0])
    pltpu.sync_copy(sqacc.at[pl.ds(0, B)], stats_out.at[cid, sid, 1])
    pltpu.sync_copy(maxacc.at[pl.ds(0, B)], stats_out.at[cid, sid, 2])

  return _sc_body


def _pool_sc(h, batch, interpret=False):
  n = h.shape[0]
  mesh = plsc.VectorSubcoreMesh(core_axis_name="c", subcore_axis_name="s",
                                num_cores=NC, num_subcores=NS)
  f = pl.kernel(
      _make_sc_body(n),
      out_type=[
          jax.ShapeDtypeStruct((NC, NS, 3, B, H), jnp.float32),
      ],
      mesh=mesh,
      interpret=interpret,
      scratch_types=[
          pltpu.VMEM((RB, H), jnp.float32),
          pltpu.VMEM((RB, H), jnp.float32),
          pltpu.VMEM((RB,), jnp.int32),
          pltpu.VMEM((RB,), jnp.int32),
          pltpu.VMEM((BD, H), jnp.float32),
          pltpu.VMEM((BD, H), jnp.float32),
          pltpu.VMEM((BD, H), jnp.float32),
          pltpu.SemaphoreType.DMA,
          pltpu.SemaphoreType.DMA,
          pltpu.SemaphoreType.DMA,
          pltpu.SemaphoreType.DMA,
      ],
  )
  return f(h, batch)[0]


def _tc_body(stats_ref, w1_ref, b1_ref, w2_ref, b2_ref, out_ref):
  st = stats_ref[...]
  s0 = jnp.sum(st[0, :, 0], axis=0)
  s1 = jnp.sum(st[1, :, 0], axis=0)
  q0 = jnp.sum(st[0, :, 1], axis=0)
  q1 = jnp.sum(st[1, :, 1], axis=0)
  m0 = jnp.max(st[0, :, 2], axis=0)
  m1 = jnp.max(st[1, :, 2], axis=0)
  ssum = jnp.concatenate([s0[:, :HC], s1[:, :HC]], axis=1)
  ssq = jnp.concatenate([q0[:, :HC], q1[:, :HC]], axis=1)
  smax = jnp.concatenate([m0[:, :HC], m1[:, :HC]], axis=1)
  count = s0[:, HC]
  safe = jnp.maximum(count, 1.0)[:, None]
  mean = ssum / safe
  var = jnp.maximum(ssq / safe - mean * mean, 0.0)
  std = jnp.sqrt(var + 1e-8)
  smax = jnp.where(count[:, None] > 0.0, smax, 0.0)
  g = jnp.concatenate([mean, smax, std], axis=1)
  hid = jax.nn.relu(
      jnp.dot(g, w1_ref[...], preferred_element_type=jnp.float32)
      + b1_ref[...])
  z = jnp.tanh(
      jnp.dot(hid, w2_ref[...], preferred_element_type=jnp.float32)
      + b2_ref[...]) * math.pi
  out_ref[...] = z


def _head_tc(stats, W1, b1, W2, b2, interpret=False):
  w2p = jnp.zeros((32, 128), jnp.float32).at[:, :8].set(W2)
  b2p = jnp.zeros((1, 128), jnp.float32).at[:, :8].set(b2)
  out = pl.pallas_call(
      _tc_body,
      out_shape=jax.ShapeDtypeStruct((B, 128), jnp.float32),
      interpret=interpret,
  )(stats, W1, b1.reshape(1, 32), w2p, b2p)
  return out[:, :8]


def kernel(h, batch, W1, b1, W2, b2):
  stats = _pool_sc(h, batch)
  return _head_tc(stats, W1, b1, W2, b2)
